# baseline (speedup 1.0000x reference)
import functools

import jax
import jax.numpy as jnp
from jax import lax
from jax.experimental import pallas as pl
from jax.experimental.pallas import tpu as pltpu
from jax.experimental.pallas import tpu_sc as plsc

N = 10000
E = 320000
R = 90
HID = 16
C = 4

NC = 2
NS = 16
NW = NC * NS
CH = 2000
SEGP = 900096

F32 = jnp.float32
I32 = jnp.int32

G = 8
NG = 12


def _mesh():
    return plsc.VectorSubcoreMesh(
        core_axis_name="c", subcore_axis_name="s", num_cores=NC, num_subcores=NS
    )


def _w_body(bhn_ref, mg_ref, c2_ref, b2_ref, ei_ref, et_ref,
            w1_ref, w2_ref, seg_ref, idx_ref):
    g = pl.program_id(0)
    w1_ref[0] = jnp.einsum(
        "km,kn->mn", bhn_ref[...], mg_ref[0], preferred_element_type=F32
    )

    @pl.when(g == 0)
    def _():
        w2_ref[...] = jnp.dot(
            c2_ref[...], b2_ref[...], preferred_element_type=F32
        )
        src = ei_ref[0, :]
        dst = ei_ref[1, :]
        et = et_ref[...]
        seg_ref[...] = dst * R + et
        idx_ref[...] = (et // G) * (N * G) + src * G + (et % G)


def _build_weights(bhn, mg, comp2, basis2r, edge_index, edge_type):
    return pl.pallas_call(
        _w_body,
        grid=(NG,),
        in_specs=[
            pl.BlockSpec((30 * HID, N), lambda g: (0, 0)),
            pl.BlockSpec((1, 30 * HID, G * HID), lambda g: (g, 0, 0)),
            pl.BlockSpec((R, 30), lambda g: (0, 0)),
            pl.BlockSpec((30, HID * C), lambda g: (0, 0)),
            pl.BlockSpec((2, E), lambda g: (0, 0)),
            pl.BlockSpec((E,), lambda g: (0,)),
        ],
        out_specs=[
            pl.BlockSpec((1, N, G * HID), lambda g: (g, 0, 0)),
            pl.BlockSpec((R, HID * C), lambda g: (0, 0)),
            pl.BlockSpec((E,), lambda g: (0,)),
            pl.BlockSpec((E,), lambda g: (0,)),
        ],
        out_shape=(
            jax.ShapeDtypeStruct((NG, N, G * HID), F32),
            jax.ShapeDtypeStruct((R, HID * C), F32),
            jax.ShapeDtypeStruct((E,), I32),
            jax.ShapeDtypeStruct((E,), I32),
        ),
        compiler_params=pltpu.CompilerParams(fuse_transposed_lhs_in_matmul=True),
    )(bhn, mg, comp2, basis2r, edge_index, edge_type)


NP = 10240
ZB = 4688


def _scale_rows(rows, normv, n_edges):
    def body(g, _):
        o = pl.multiple_of(g * 16, 16)
        nv16 = normv[pl.ds(o, 16)]
        for l in range(16):
            rows[o + l] = rows[o + l] * nv16[l]
        return 0

    lax.fori_loop(0, n_edges // 16, body, 0)


def _l1_fused_sc(seg, idx, dst, ones_c, table):
    @functools.partial(
        pl.kernel,
        out_type=(
            jax.ShapeDtypeStruct((E,), F32),
            jax.ShapeDtypeStruct((NC, NP, HID), F32),
        ),
        mesh=_mesh(),
        compiler_params=pltpu.CompilerParams(use_tc_tiling_on_sc=False),
        scratch_types=[
            pltpu.VMEM_SHARED((SEGP,), F32),
            pltpu.VMEM_SHARED((NP, HID), F32),
            pltpu.VMEM((ZB,), F32),
            pltpu.VMEM((NP // NS, HID), F32),
            pltpu.VMEM((CH,), I32),
            pltpu.VMEM((CH,), F32),
            pltpu.VMEM((CH,), F32),
            pltpu.VMEM((CH,), F32),
            pltpu.VMEM((CH,), I32),
            pltpu.VMEM((CH,), I32),
            pltpu.VMEM((CH, HID), F32),
            pltpu.SemaphoreType.DMA,
        ],
    )
    def k(seg_hbm, idx_hbm, dst_hbm, ones_hbm, tab_hbm, norm_hbm, hp_hbm,
          counts, acc, zbuf, stage, segv, onesv, cv, nv, idxv, dstv,
          rows, sem):
        c = lax.axis_index("c")
        s = lax.axis_index("s")
        wid = s * NC + c
        zr = NP // NS
        rb = pl.multiple_of(s * zr, 8)

        def zb_body(k2, _):
            o = pl.multiple_of(k2 * 16, 16)
            zbuf[pl.ds(o, 16)] = jnp.zeros((16,), F32)
            return 0

        lax.fori_loop(0, ZB // 16, zb_body, 0)

        def zs_body(k2, _):
            stage[k2] = jnp.zeros((16,), F32)
            return 0

        lax.fori_loop(0, zr, zs_body, 0)
        zch = SEGP // NS
        for i in range(zch // ZB):
            zo = pl.multiple_of(s * zch + i * ZB, 8)
            pltpu.sync_copy(zbuf, counts.at[pl.ds(zo, ZB)])
        pltpu.sync_copy(stage, acc.at[pl.ds(rb, zr), :])
        pltpu.sync_copy(ones_hbm, onesv)
        plsc.subcore_barrier()
        eps = E // NS
        for j in range(eps // CH):
            base1 = pl.multiple_of(s * eps + j * CH, 8)
            pltpu.sync_copy(seg_hbm.at[pl.ds(base1, CH)], segv)
            pltpu.sync_copy(onesv, counts.at[segv], add=True)
        plsc.subcore_barrier()
        epw = E // NW
        for j in range(epw // CH):
            base = pl.multiple_of(wid * epw + j * CH, 8)
            pltpu.sync_copy(idx_hbm.at[pl.ds(base, CH)], idxv)
            gat = pltpu.async_copy(tab_hbm.at[idxv], rows, sem)
            pltpu.sync_copy(seg_hbm.at[pl.ds(base, CH)], segv)
            pltpu.sync_copy(dst_hbm.at[pl.ds(base, CH)], dstv)
            pltpu.sync_copy(counts.at[segv], cv)

            def nbody(k2, _):
                o = pl.multiple_of(k2 * 16, 16)
                v = cv[pl.ds(o, 16)]
                nv[pl.ds(o, 16)] = 1.0 / jnp.maximum(v, 1.0)
                return 0

            lax.fori_loop(0, CH // 16, nbody, 0)
            pltpu.sync_copy(nv, norm_hbm.at[pl.ds(base, CH)])
            gat.wait()
            _scale_rows(rows, nv, CH)
            pltpu.sync_copy(rows, acc.at[dstv], add=True)
        plsc.subcore_barrier()
        pltpu.sync_copy(acc.at[pl.ds(rb, zr), :], stage)
        pltpu.sync_copy(stage, hp_hbm.at[c, pl.ds(rb, zr), :])

    return k(seg, idx, dst, ones_c, table)


def _edge_pass_sc(table, idx, dst, norm):
    @functools.partial(
        pl.kernel,
        out_type=jax.ShapeDtypeStruct((NC, NP, HID), F32),
        mesh=_mesh(),
        compiler_params=pltpu.CompilerParams(use_tc_tiling_on_sc=False),
        scratch_types=[
            pltpu.VMEM_SHARED((NP, HID), F32),
            pltpu.VMEM((NP // NS, HID), F32),
            pltpu.VMEM((CH,), I32),
            pltpu.VMEM((CH,), I32),
            pltpu.VMEM((CH,), F32),
            pltpu.VMEM((CH, HID), F32),
            pltpu.SemaphoreType.DMA,
        ],
    )
    def k(tab_hbm, idx_hbm, dst_hbm, norm_hbm, out_hbm,
          acc, stage, idxv, dstv, normv, rows, sem):
        c = lax.axis_index("c")
        s = lax.axis_index("s")
        wid = s * NC + c
        zr = NP // NS
        rb = pl.multiple_of(s * zr, 8)

        def zbody(k2, _):
            stage[k2] = jnp.zeros((16,), F32)
            return 0

        lax.fori_loop(0, zr, zbody, 0)
        pltpu.sync_copy(stage, acc.at[pl.ds(rb, zr), :])
        plsc.subcore_barrier()
        epw = E // NW
        for j in range(epw // CH):
            base = pl.multiple_of(wid * epw + j * CH, 8)
            pltpu.sync_copy(idx_hbm.at[pl.ds(base, CH)], idxv)
            gat = pltpu.async_copy(tab_hbm.at[idxv], rows, sem)
            pltpu.sync_copy(dst_hbm.at[pl.ds(base, CH)], dstv)
            pltpu.sync_copy(norm_hbm.at[pl.ds(base, CH)], normv)
            gat.wait()
            _scale_rows(rows, normv, CH)
            pltpu.sync_copy(rows, acc.at[dstv], add=True)
        plsc.subcore_barrier()
        pltpu.sync_copy(acc.at[pl.ds(rb, zr), :], stage)
        pltpu.sync_copy(stage, out_hbm.at[c, pl.ds(rb, zr), :])

    return k(table, idx, dst, norm)


def _m2_body(hp_ref, r1_ref, b1_ref, r2_ref, b2_ref, w2_ref, m2_ref, hr_ref):
    g = pl.program_id(0)
    t = hp_ref[0] + hp_ref[1]
    h = jnp.maximum(t[:N] + r1_ref[...] + b1_ref[...], 0.0)
    m2_ref[0] = jnp.dot(h, w2_ref[0], preferred_element_type=F32)

    @pl.when(g == 0)
    def _():
        hr_ref[...] = (
            jnp.dot(h, r2_ref[...], preferred_element_type=F32) + b2_ref[...]
        )


def _m2_build(hp, root1, bias1, root2, bias2, w2cat):
    return pl.pallas_call(
        _m2_body,
        grid=(NG,),
        in_specs=[
            pl.BlockSpec((NC, NP, HID), lambda g: (0, 0, 0)),
            pl.BlockSpec((N, HID), lambda g: (0, 0)),
            pl.BlockSpec((1, HID), lambda g: (0, 0)),
            pl.BlockSpec((HID, C), lambda g: (0, 0)),
            pl.BlockSpec((1, C), lambda g: (0, 0)),
            pl.BlockSpec((1, HID, G * HID), lambda g: (g, 0, 0)),
        ],
        out_specs=[
            pl.BlockSpec((1, N, G * HID), lambda g: (g, 0, 0)),
            pl.BlockSpec((N, C), lambda g: (0, 0)),
        ],
        out_shape=(
            jax.ShapeDtypeStruct((NG, N, G * HID), F32),
            jax.ShapeDtypeStruct((N, C), F32),
        ),
    )(hp, root1, bias1.reshape(1, HID), root2, bias2.reshape(1, C), w2cat)


def _out_body(op_ref, hr_ref, o_ref):
    t = op_ref[0] + op_ref[1]
    x = t[:N, 0:C] + hr_ref[...]
    m = jnp.max(x, axis=1, keepdims=True)
    z = x - m
    lse = jnp.log(jnp.sum(jnp.exp(z), axis=1, keepdims=True))
    o_ref[...] = z - lse


def _final(op, hroot):
    return pl.pallas_call(
        _out_body,
        out_shape=jax.ShapeDtypeStruct((N, C), F32),
    )(op, hroot)


def kernel(edge_index, edge_type, basis1, comp1, root1, bias1,
           basis2, comp2, root2, bias2):
    basis2r = basis2.reshape(30, HID * C)
    bhn = basis1.transpose(0, 2, 1).reshape(30 * HID, N).astype(jnp.bfloat16)
    c1p = jnp.pad(comp1, ((0, NG * G - R), (0, 0))).reshape(NG, G, 30)
    eye = jnp.eye(HID, dtype=F32)
    mg = jnp.einsum("gkb,ph->gbpkh", c1p, eye).reshape(
        NG, 30 * HID, G * HID).astype(jnp.bfloat16)
    w1m, w2m, seg, idxg = _build_weights(
        bhn, mg, comp2, basis2r, edge_index, edge_type)
    w1 = w1m.reshape(NG * N * G, HID)
    w2p = jnp.pad(w2m.reshape(R, HID, C),
                  ((0, NG * G - R), (0, 0), (0, HID - C)))
    w2cat = w2p.reshape(NG, G, HID, HID).transpose(0, 2, 1, 3).reshape(
        NG, HID, G * HID)

    ones_c = jnp.ones((CH,), F32)
    dst = edge_index[1]

    norm, hp = _l1_fused_sc(seg, idxg, dst, ones_c, w1)
    m2, hroot = _m2_build(hp, root1, bias1, root2, bias2, w2cat)
    op = _edge_pass_sc(m2.reshape(NG * N * G, HID), idxg, dst, norm)
    return _final(op, hroot)


# speedup vs baseline: 1.0117x; 1.0117x over previous
import functools

import jax
import jax.numpy as jnp
from jax import lax
from jax.experimental import pallas as pl
from jax.experimental.pallas import tpu as pltpu
from jax.experimental.pallas import tpu_sc as plsc

N = 10000
E = 320000
R = 90
HID = 16
C = 4

NC = 2
NS = 16
NW = NC * NS
CH = 2000
SEGP = 900096

F32 = jnp.float32
I32 = jnp.int32

G = 8
NG = 12


def _mesh():
    return plsc.VectorSubcoreMesh(
        core_axis_name="c", subcore_axis_name="s", num_cores=NC, num_subcores=NS
    )


def _w_body(bhn_ref, mg_ref, c2_ref, b2_ref, ei_ref, et_ref,
            w1_ref, w2_ref, seg_ref, idx_ref):
    g = pl.program_id(0)
    w1_ref[0] = jnp.einsum(
        "km,kn->mn", bhn_ref[...], mg_ref[0], preferred_element_type=F32
    )

    @pl.when(g == 0)
    def _():
        w2_ref[...] = jnp.dot(
            c2_ref[...], b2_ref[...], preferred_element_type=F32
        )
        src = ei_ref[0, :]
        dst = ei_ref[1, :]
        et = et_ref[...]
        seg_ref[...] = dst * R + et
        idx_ref[...] = (et // G) * (N * G) + src * G + (et % G)


def _build_weights(bhn, mg, comp2, basis2r, edge_index, edge_type):
    return pl.pallas_call(
        _w_body,
        grid=(NG,),
        in_specs=[
            pl.BlockSpec((30 * HID, N), lambda g: (0, 0)),
            pl.BlockSpec((1, 30 * HID, G * HID), lambda g: (g, 0, 0)),
            pl.BlockSpec((R, 30), lambda g: (0, 0)),
            pl.BlockSpec((30, HID * C), lambda g: (0, 0)),
            pl.BlockSpec((2, E), lambda g: (0, 0)),
            pl.BlockSpec((E,), lambda g: (0,)),
        ],
        out_specs=[
            pl.BlockSpec((1, N, G * HID), lambda g: (g, 0, 0)),
            pl.BlockSpec((R, HID * C), lambda g: (0, 0)),
            pl.BlockSpec((E,), lambda g: (0,)),
            pl.BlockSpec((E,), lambda g: (0,)),
        ],
        out_shape=(
            jax.ShapeDtypeStruct((NG, N, G * HID), F32),
            jax.ShapeDtypeStruct((R, HID * C), F32),
            jax.ShapeDtypeStruct((E,), I32),
            jax.ShapeDtypeStruct((E,), I32),
        ),
        compiler_params=pltpu.CompilerParams(fuse_transposed_lhs_in_matmul=True),
    )(bhn, mg, comp2, basis2r, edge_index, edge_type)


NP = 10240
ZB = 4688


def _scale_rows(rows, normv, n_edges):
    def body(g, _):
        o = pl.multiple_of(g * 16, 16)
        nv16 = normv[pl.ds(o, 16)]
        for l in range(16):
            rows[o + l] = rows[o + l] * nv16[l]
        return 0

    lax.fori_loop(0, n_edges // 16, body, 0)


def _l1_fused_sc(seg, idx, dst, ones_c, table):
    @functools.partial(
        pl.kernel,
        out_type=(
            jax.ShapeDtypeStruct((E,), F32),
            jax.ShapeDtypeStruct((NC, NP, HID), F32),
        ),
        mesh=_mesh(),
        compiler_params=pltpu.CompilerParams(use_tc_tiling_on_sc=False),
        scratch_types=[
            pltpu.VMEM_SHARED((SEGP,), F32),
            pltpu.VMEM_SHARED((NP, HID), F32),
            pltpu.VMEM((ZB,), F32),
            pltpu.VMEM((NP // NS, HID), F32),
            pltpu.VMEM((CH,), I32),
            pltpu.VMEM((CH,), F32),
            pltpu.VMEM((CH,), F32),
            pltpu.VMEM((CH,), F32),
            pltpu.VMEM((CH,), I32),
            pltpu.VMEM((CH,), I32),
            pltpu.VMEM((CH, HID), F32),
            pltpu.SemaphoreType.DMA,
        ],
    )
    def k(seg_hbm, idx_hbm, dst_hbm, ones_hbm, tab_hbm, norm_hbm, hp_hbm,
          counts, acc, zbuf, stage, segv, onesv, cv, nv, idxv, dstv,
          rows, sem):
        c = lax.axis_index("c")
        s = lax.axis_index("s")
        wid = s * NC + c
        zr = NP // NS
        rb = pl.multiple_of(s * zr, 8)

        def zb_body(k2, _):
            o = pl.multiple_of(k2 * 16, 16)
            zbuf[pl.ds(o, 16)] = jnp.zeros((16,), F32)
            return 0

        lax.fori_loop(0, ZB // 16, zb_body, 0)

        def zs_body(k2, _):
            stage[k2] = jnp.zeros((16,), F32)
            return 0

        lax.fori_loop(0, zr, zs_body, 0)
        zch = SEGP // NS
        for i in range(zch // ZB):
            zo = pl.multiple_of(s * zch + i * ZB, 8)
            pltpu.sync_copy(zbuf, counts.at[pl.ds(zo, ZB)])
        pltpu.sync_copy(stage, acc.at[pl.ds(rb, zr), :])
        pltpu.sync_copy(ones_hbm, onesv)
        plsc.subcore_barrier()
        eps = E // NS
        for j in range(eps // CH):
            base1 = pl.multiple_of(s * eps + j * CH, 8)
            pltpu.sync_copy(seg_hbm.at[pl.ds(base1, CH)], segv)
            pltpu.sync_copy(onesv, counts.at[segv], add=True)
        plsc.subcore_barrier()
        epw = E // NW
        for j in range(epw // CH):
            base = pl.multiple_of(wid * epw + j * CH, 8)
            pltpu.sync_copy(idx_hbm.at[pl.ds(base, CH)], idxv)
            gat = pltpu.async_copy(tab_hbm.at[idxv], rows, sem)
            pltpu.sync_copy(seg_hbm.at[pl.ds(base, CH)], segv)
            pltpu.sync_copy(dst_hbm.at[pl.ds(base, CH)], dstv)
            pltpu.sync_copy(counts.at[segv], cv)

            def nbody(k2, _):
                o = pl.multiple_of(k2 * 16, 16)
                v = cv[pl.ds(o, 16)]
                nv[pl.ds(o, 16)] = 1.0 / jnp.maximum(v, 1.0)
                return 0

            lax.fori_loop(0, CH // 16, nbody, 0)
            pltpu.sync_copy(nv, norm_hbm.at[pl.ds(base, CH)])
            gat.wait()
            _scale_rows(rows, nv, CH)
            pltpu.sync_copy(rows, acc.at[dstv], add=True)
        plsc.subcore_barrier()
        pltpu.sync_copy(acc.at[pl.ds(rb, zr), :], stage)
        pltpu.sync_copy(stage, hp_hbm.at[c, pl.ds(rb, zr), :])

    return k(seg, idx, dst, ones_c, table)


def _edge_pass_sc(table, idx, dst, norm):
    @functools.partial(
        pl.kernel,
        out_type=jax.ShapeDtypeStruct((NC, NP, HID), F32),
        mesh=_mesh(),
        compiler_params=pltpu.CompilerParams(use_tc_tiling_on_sc=False),
        scratch_types=[
            pltpu.VMEM_SHARED((NP, HID), F32),
            pltpu.VMEM((NP // NS, HID), F32),
            pltpu.VMEM((CH,), I32),
            pltpu.VMEM((CH,), I32),
            pltpu.VMEM((CH,), F32),
            pltpu.VMEM((CH, HID), F32),
            pltpu.SemaphoreType.DMA,
        ],
    )
    def k(tab_hbm, idx_hbm, dst_hbm, norm_hbm, out_hbm,
          acc, stage, idxv, dstv, normv, rows, sem):
        c = lax.axis_index("c")
        s = lax.axis_index("s")
        wid = s * NC + c
        zr = NP // NS
        rb = pl.multiple_of(s * zr, 8)

        def zbody(k2, _):
            stage[k2] = jnp.zeros((16,), F32)
            return 0

        lax.fori_loop(0, zr, zbody, 0)
        pltpu.sync_copy(stage, acc.at[pl.ds(rb, zr), :])
        plsc.subcore_barrier()
        epw = E // NW
        for j in range(epw // CH):
            base = pl.multiple_of(wid * epw + j * CH, 8)
            pltpu.sync_copy(idx_hbm.at[pl.ds(base, CH)], idxv)
            gat = pltpu.async_copy(tab_hbm.at[idxv], rows, sem)
            pltpu.sync_copy(dst_hbm.at[pl.ds(base, CH)], dstv)
            pltpu.sync_copy(norm_hbm.at[pl.ds(base, CH)], normv)
            gat.wait()
            _scale_rows(rows, normv, CH)
            pltpu.sync_copy(rows, acc.at[dstv], add=True)
        plsc.subcore_barrier()
        pltpu.sync_copy(acc.at[pl.ds(rb, zr), :], stage)
        pltpu.sync_copy(stage, out_hbm.at[c, pl.ds(rb, zr), :])

    return k(table, idx, dst, norm)


def _m2_body(hp_ref, r1_ref, b1_ref, r2_ref, b2_ref, w2_ref, m2_ref, hr_ref):
    g = pl.program_id(0)
    t = hp_ref[0] + hp_ref[1]
    h = jnp.maximum(t[:N] + r1_ref[...] + b1_ref[...], 0.0)
    m2_ref[0] = jnp.dot(h, w2_ref[0], preferred_element_type=F32)

    @pl.when(g == 0)
    def _():
        hr_ref[...] = (
            jnp.dot(h, r2_ref[...], preferred_element_type=F32) + b2_ref[...]
        )


def _m2_build(hp, root1, bias1, root2, bias2, w2cat):
    return pl.pallas_call(
        _m2_body,
        grid=(NG,),
        in_specs=[
            pl.BlockSpec((NC, NP, HID), lambda g: (0, 0, 0)),
            pl.BlockSpec((N, HID), lambda g: (0, 0)),
            pl.BlockSpec((1, HID), lambda g: (0, 0)),
            pl.BlockSpec((HID, C), lambda g: (0, 0)),
            pl.BlockSpec((1, C), lambda g: (0, 0)),
            pl.BlockSpec((1, HID, G * HID), lambda g: (g, 0, 0)),
        ],
        out_specs=[
            pl.BlockSpec((1, N, G * HID), lambda g: (g, 0, 0)),
            pl.BlockSpec((N, C), lambda g: (0, 0)),
        ],
        out_shape=(
            jax.ShapeDtypeStruct((NG, N, G * HID), F32),
            jax.ShapeDtypeStruct((N, C), F32),
        ),
    )(hp, root1, bias1.reshape(1, HID), root2, bias2.reshape(1, C), w2cat)


def _out_body(op_ref, hr_ref, o_ref):
    t = op_ref[0] + op_ref[1]
    x = t[:N, 0:C] + hr_ref[...]
    m = jnp.max(x, axis=1, keepdims=True)
    z = x - m
    lse = jnp.log(jnp.sum(jnp.exp(z), axis=1, keepdims=True))
    o_ref[...] = z - lse


def _final(op, hroot):
    return pl.pallas_call(
        _out_body,
        out_shape=jax.ShapeDtypeStruct((N, C), F32),
    )(op, hroot)


def kernel(edge_index, edge_type, basis1, comp1, root1, bias1,
           basis2, comp2, root2, bias2):
    basis2r = basis2.reshape(30, HID * C)
    bhn = basis1.transpose(0, 2, 1).reshape(30 * HID, N)
    c1p = jnp.pad(comp1, ((0, NG * G - R), (0, 0))).reshape(NG, G, 30)
    eye = jnp.eye(HID, dtype=F32)
    mg = jnp.einsum("gkb,ph->gbpkh", c1p, eye).reshape(
        NG, 30 * HID, G * HID)
    w1m, w2m, seg, idxg = _build_weights(
        bhn, mg, comp2, basis2r, edge_index, edge_type)
    w1 = w1m.reshape(NG * N * G, HID)
    w2p = jnp.pad(w2m.reshape(R, HID, C),
                  ((0, NG * G - R), (0, 0), (0, HID - C)))
    w2cat = w2p.reshape(NG, G, HID, HID).transpose(0, 2, 1, 3).reshape(
        NG, HID, G * HID)

    ones_c = jnp.ones((CH,), F32)
    dst = edge_index[1]

    norm, hp = _l1_fused_sc(seg, idxg, dst, ones_c, w1)
    m2, hroot = _m2_build(hp, root1, bias1, root2, bias2, w2cat)
    op = _edge_pass_sc(m2.reshape(NG * N * G, HID), idxg, dst, norm)
    return _final(op, hroot)
